# baseline (speedup 1.0000x reference)
import jax
import jax.numpy as jnp
from jax import lax
from jax.experimental import pallas as pl
from jax.experimental.pallas import tpu as pltpu
from jax.experimental.pallas import tpu_sc as plsc

F = 26
V = 100000
D = 32
B = 16384

NC = 2
NS = 16
NW = NC * NS
NTASK = F * D
TPW = NTASK // NW
GCH = 4096
NG = B // GCH
L = 16


def _embed_body(tab_hbm, cx_hbm, out_hbm, row_v, idx_v, out_a, out_b, sem_a, sem_b):
    wid = lax.axis_index("s") * NC + lax.axis_index("c")
    base = wid * TPW

    def gather_chunk(g, out_v):
        @plsc.parallel_loop(0, GCH // L, unroll=16)
        def grp(j):
            vec = idx_v[pl.ds(g * GCH + j * L, L)]
            out_v[pl.ds(j * L, L)] = plsc.load_gather(row_v, [vec])

    def chunks(f, d, drain_from):
        for g in range(NG):
            buf, sem = (out_a, sem_a) if g % 2 == 0 else (out_b, sem_b)
            dst = out_hbm.at[f, d, pl.ds(g * GCH, GCH)]
            if g >= drain_from:
                pltpu.make_async_copy(dst, buf, sem).wait()
            gather_chunk(g, buf)
            pltpu.async_copy(buf, dst, sem)

    f0 = base // D
    d0 = base % D
    pltpu.sync_copy(cx_hbm.at[f0], idx_v)
    pltpu.sync_copy(tab_hbm.at[f0, d0], row_v)
    chunks(f0, d0, drain_from=2)

    def task(t, carry):
        tid = base + t
        f = tid // D
        d = tid % D
        pltpu.sync_copy(tab_hbm.at[f, d], row_v)
        @pl.when(d == 0)
        def _():
            pltpu.sync_copy(cx_hbm.at[f], idx_v)

        chunks(f, d, drain_from=0)
        return carry

    lax.fori_loop(1, TPW, task, 0)
    pltpu.make_async_copy(out_hbm.at[f0, d0, pl.ds(0, GCH)], out_a, sem_a).wait()
    pltpu.make_async_copy(out_hbm.at[f0, d0, pl.ds(GCH, GCH)], out_b, sem_b).wait()


@jax.jit
def kernel(cx, tables):
    cx_t = cx.T.astype(jnp.int32)
    tab_t = jnp.transpose(tables, (0, 2, 1))
    run = pl.kernel(
        _embed_body,
        out_type=jax.ShapeDtypeStruct((F, D, B), jnp.float32),
        mesh=plsc.VectorSubcoreMesh(core_axis_name="c", subcore_axis_name="s"),
        scratch_types=[
            pltpu.VMEM((V,), jnp.float32),
            pltpu.VMEM((B,), jnp.int32),
            pltpu.VMEM((GCH,), jnp.float32),
            pltpu.VMEM((GCH,), jnp.float32),
            pltpu.SemaphoreType.DMA,
            pltpu.SemaphoreType.DMA,
        ],
        compiler_params=pltpu.CompilerParams(use_tc_tiling_on_sc=True, needs_layout_passes=False),
    )
    out_t = run(tab_t, cx_t)
    return jnp.transpose(out_t, (2, 0, 1))


# speedup vs baseline: 1.0045x; 1.0045x over previous
import jax
import jax.numpy as jnp
from jax import lax
from jax.experimental import pallas as pl
from jax.experimental.pallas import tpu as pltpu
from jax.experimental.pallas import tpu_sc as plsc

F = 26
V = 100000
D = 32
B = 16384

NC = 2
NS = 16
NW = NC * NS
NTASK = F * D
TPW = NTASK // NW
GCH = 4096
NG = B // GCH
L = 16


def _embed_body(tab_hbm, cx_hbm, out_hbm, row_v, idx_v, out_a, out_b, sem_a, sem_b):
    wid = lax.axis_index("s") * NC + lax.axis_index("c")
    base = wid * TPW

    def gather_chunk(g, out_v):
        @plsc.parallel_loop(0, GCH // L, unroll=8)
        def grp(j):
            vec = idx_v[pl.ds(g * GCH + j * L, L)]
            out_v[pl.ds(j * L, L)] = plsc.load_gather(row_v, [vec])

    def chunks(f, d, drain_from):
        for g in range(NG):
            buf, sem = (out_a, sem_a) if g % 2 == 0 else (out_b, sem_b)
            dst = out_hbm.at[f, d, pl.ds(g * GCH, GCH)]
            if g >= drain_from:
                pltpu.make_async_copy(dst, buf, sem).wait()
            gather_chunk(g, buf)
            pltpu.async_copy(buf, dst, sem)

    f0 = base // D
    d0 = base % D
    pltpu.sync_copy(cx_hbm.at[f0], idx_v)
    pltpu.sync_copy(tab_hbm.at[f0, d0], row_v)
    chunks(f0, d0, drain_from=2)

    def task(t, carry):
        tid = base + t
        f = tid // D
        d = tid % D
        pltpu.sync_copy(tab_hbm.at[f, d], row_v)
        @pl.when(d == 0)
        def _():
            pltpu.sync_copy(cx_hbm.at[f], idx_v)

        chunks(f, d, drain_from=0)
        return carry

    lax.fori_loop(1, TPW, task, 0)
    pltpu.make_async_copy(out_hbm.at[f0, d0, pl.ds(0, GCH)], out_a, sem_a).wait()
    pltpu.make_async_copy(out_hbm.at[f0, d0, pl.ds(GCH, GCH)], out_b, sem_b).wait()


@jax.jit
def kernel(cx, tables):
    cx_t = cx.T.astype(jnp.int32)
    tab_t = jnp.transpose(tables, (0, 2, 1))
    run = pl.kernel(
        _embed_body,
        out_type=jax.ShapeDtypeStruct((F, D, B), jnp.float32),
        mesh=plsc.VectorSubcoreMesh(core_axis_name="c", subcore_axis_name="s"),
        scratch_types=[
            pltpu.VMEM((V,), jnp.float32),
            pltpu.VMEM((B,), jnp.int32),
            pltpu.VMEM((GCH,), jnp.float32),
            pltpu.VMEM((GCH,), jnp.float32),
            pltpu.SemaphoreType.DMA,
            pltpu.SemaphoreType.DMA,
        ],
        compiler_params=pltpu.CompilerParams(use_tc_tiling_on_sc=True, needs_layout_passes=False),
    )
    out_t = run(tab_t, cx_t)
    return jnp.transpose(out_t, (2, 0, 1))
